# baseline (speedup 1.0000x reference)
import functools

import jax
import jax.numpy as jnp
from jax.experimental import pallas as pl
import jax.experimental.pallas.tpu as pltpu

_TOP_K = 8
_NEG = -1e30
_NCH = 4


def _body(facts_ref, beta_hbm, alT_ref, rs_ref, W_hbm, gamma_ref, lnb_ref,
          out_ref, beta_vmem, W_vmem, bsems, wsem):
    R = beta_hbm.shape[0]
    ch = R // _NCH
    for c in range(_NCH):
        pltpu.make_async_copy(beta_hbm.at[pl.ds(c * ch, ch), :],
                              beta_vmem.at[pl.ds(c * ch, ch), :],
                              bsems.at[c]).start()
    wcopy = pltpu.make_async_copy(W_hbm, W_vmem, wsem)
    wcopy.start()

    facts = facts_ref[...]
    B = facts.shape[0]
    log_f = jnp.log(jnp.maximum(facts, 1e-30))
    log_1mf = jnp.log(jnp.maximum(1.0 - facts, 1e-30))
    logs = jnp.concatenate([log_f, log_1mf], axis=0).astype(jnp.bfloat16)

    dn = (((1,), (1,)), ((), ()))
    mm = functools.partial(jax.lax.dot_general, dimension_numbers=dn,
                           preferred_element_type=jnp.float32,
                           precision=jax.lax.Precision.HIGHEST)
    mm_bf = functools.partial(jax.lax.dot_general, dimension_numbers=dn,
                              preferred_element_type=jnp.float32)

    prod_parts, sum_parts, cnt_parts = [], [], []
    for c in range(_NCH):
        pltpu.make_async_copy(beta_hbm.at[pl.ds(c * ch, ch), :],
                              beta_vmem.at[pl.ds(c * ch, ch), :],
                              bsems.at[c]).wait()
        beta_c = beta_vmem[pl.ds(c * ch, ch), :]
        mask = jnp.where(beta_c > 0.0, 1.0, 0.0)
        prod_parts.append(mm_bf(logs, mask.astype(jnp.bfloat16)))
        sum_parts.append(mm(facts, mask))
        cnt_parts.append(jnp.sum(mask, axis=1)[None, :])

    prods = jnp.exp(jnp.concatenate(prod_parts, axis=1))
    and_agg = prods[:B]
    or_agg = 1.0 - prods[B:]
    s_sum = jnp.concatenate(sum_parts, axis=1)
    cnt = jnp.concatenate(cnt_parts, axis=1) + 1e-08
    k_of_n = s_sum / cnt

    w = jax.nn.softmax(alT_ref[...], axis=0)
    mixed = (w[0][None, :] * and_agg + w[1][None, :] * or_agg
             + w[2][None, :] * k_of_n + w[3][None, :] * (1.0 - k_of_n))
    act = mixed * jax.nn.sigmoid(rs_ref[...])

    iota = jax.lax.broadcasted_iota(jnp.int32, act.shape, 1)
    a = act
    gate = jnp.zeros_like(act)
    for _ in range(_TOP_K):
        m = jnp.max(a, axis=1, keepdims=True)
        idx = jnp.min(jnp.where(a == m, iota, act.shape[1]), axis=1,
                      keepdims=True)
        sel = iota == idx
        gate = jnp.where(sel, 1.0, gate)
        a = jnp.where(sel, _NEG, a)

    wcopy.wait()
    pre = mm(facts, W_vmem[...]) + act * gate
    mu = jnp.mean(pre, axis=1, keepdims=True)
    var = jnp.mean(pre * pre, axis=1, keepdims=True) - mu * mu
    out_ref[...] = ((pre - mu) * jax.lax.rsqrt(var + 1e-05)
                    * gamma_ref[...] + lnb_ref[...])


def kernel(facts, beta, aggregator_logits, rule_strength_raw, W, gamma,
           ln_beta):
    B, D = facts.shape
    R, _ = beta.shape
    return pl.pallas_call(
        _body,
        in_specs=[
            pl.BlockSpec(memory_space=pltpu.MemorySpace.VMEM),
            pl.BlockSpec(memory_space=pltpu.MemorySpace.HBM),
            pl.BlockSpec(memory_space=pltpu.MemorySpace.VMEM),
            pl.BlockSpec(memory_space=pltpu.MemorySpace.VMEM),
            pl.BlockSpec(memory_space=pltpu.MemorySpace.HBM),
            pl.BlockSpec(memory_space=pltpu.MemorySpace.VMEM),
            pl.BlockSpec(memory_space=pltpu.MemorySpace.VMEM),
        ],
        out_specs=pl.BlockSpec(memory_space=pltpu.MemorySpace.VMEM),
        out_shape=jax.ShapeDtypeStruct((B, R), jnp.float32),
        scratch_shapes=[
            pltpu.VMEM((R, D), jnp.float32),
            pltpu.VMEM((R, D), jnp.float32),
            pltpu.SemaphoreType.DMA((_NCH,)),
            pltpu.SemaphoreType.DMA,
        ],
    )(facts, beta, aggregator_logits.T, rule_strength_raw[None, :], W,
      gamma[None, :], ln_beta[None, :])


# speedup vs baseline: 1.1326x; 1.1326x over previous
import functools

import jax
import jax.numpy as jnp
from jax.experimental import pallas as pl
import jax.experimental.pallas.tpu as pltpu

_TOP_K = 8
_NEG = -1e30


def _body(facts_ref, beta_hbm, alT_ref, rs_ref, W_hbm, gamma_ref, lnb_ref,
          out_ref, beta_vmem, W_vmem, bsem, wsem):
    bcopy = pltpu.make_async_copy(beta_hbm, beta_vmem, bsem)
    bcopy.start()
    wcopy = pltpu.make_async_copy(W_hbm, W_vmem, wsem)
    wcopy.start()

    facts = facts_ref[...]
    B = facts.shape[0]
    log_f = jnp.log(jnp.maximum(facts, 1e-30))
    log_1mf = jnp.log(jnp.maximum(1.0 - facts, 1e-30))
    logs = jnp.concatenate([log_f, log_1mf], axis=0).astype(jnp.bfloat16)

    dn = (((1,), (1,)), ((), ()))
    mm = functools.partial(jax.lax.dot_general, dimension_numbers=dn,
                           preferred_element_type=jnp.float32,
                           precision=jax.lax.Precision.HIGHEST)
    mm_bf = functools.partial(jax.lax.dot_general, dimension_numbers=dn,
                              preferred_element_type=jnp.float32)

    bcopy.wait()
    mask = jnp.where(beta_vmem[...] > 0.0, 1.0, 0.0)
    prods = jnp.exp(mm_bf(logs, mask.astype(jnp.bfloat16)))
    and_agg = prods[:B]
    or_agg = 1.0 - prods[B:]
    s_sum = mm(facts, mask)
    cnt = jnp.sum(mask, axis=1)[None, :] + 1e-08
    k_of_n = s_sum / cnt

    w = jax.nn.softmax(alT_ref[...], axis=0)
    mixed = (w[0][None, :] * and_agg + w[1][None, :] * or_agg
             + w[2][None, :] * k_of_n + w[3][None, :] * (1.0 - k_of_n))
    act = mixed * jax.nn.sigmoid(rs_ref[...])

    iota = jax.lax.broadcasted_iota(jnp.int32, act.shape, 1)
    a = act
    gate = jnp.zeros_like(act)
    for _ in range(_TOP_K):
        m = jnp.max(a, axis=1, keepdims=True)
        idx = jnp.min(jnp.where(a == m, iota, act.shape[1]), axis=1,
                      keepdims=True)
        sel = iota == idx
        gate = jnp.where(sel, 1.0, gate)
        a = jnp.where(sel, _NEG, a)

    wcopy.wait()
    pre = mm(facts, W_vmem[...]) + act * gate
    mu = jnp.mean(pre, axis=1, keepdims=True)
    var = jnp.mean(pre * pre, axis=1, keepdims=True) - mu * mu
    out_ref[...] = ((pre - mu) * jax.lax.rsqrt(var + 1e-05)
                    * gamma_ref[...] + lnb_ref[...])


def kernel(facts, beta, aggregator_logits, rule_strength_raw, W, gamma,
           ln_beta):
    B, D = facts.shape
    R, _ = beta.shape
    return pl.pallas_call(
        _body,
        in_specs=[
            pl.BlockSpec(memory_space=pltpu.MemorySpace.VMEM),
            pl.BlockSpec(memory_space=pltpu.MemorySpace.HBM),
            pl.BlockSpec(memory_space=pltpu.MemorySpace.VMEM),
            pl.BlockSpec(memory_space=pltpu.MemorySpace.VMEM),
            pl.BlockSpec(memory_space=pltpu.MemorySpace.HBM),
            pl.BlockSpec(memory_space=pltpu.MemorySpace.VMEM),
            pl.BlockSpec(memory_space=pltpu.MemorySpace.VMEM),
        ],
        out_specs=pl.BlockSpec(memory_space=pltpu.MemorySpace.VMEM),
        out_shape=jax.ShapeDtypeStruct((B, R), jnp.float32),
        scratch_shapes=[
            pltpu.VMEM((R, D), jnp.float32),
            pltpu.VMEM((R, D), jnp.float32),
            pltpu.SemaphoreType.DMA,
            pltpu.SemaphoreType.DMA,
        ],
    )(facts, beta, aggregator_logits.T, rule_strength_raw[None, :], W,
      gamma[None, :], ln_beta[None, :])
